# baseline (speedup 1.0000x reference)
import functools

import jax
import jax.numpy as jnp
from jax import lax
from jax.experimental import pallas as pl
from jax.experimental.pallas import tpu as pltpu
from jax.experimental.pallas import tpu_sc as plsc

N = 10000
E = 320000
D = 128
G = 96
DW = 16
C = 128
NIB = 4
NSUB = 16
PER_TILE = 20480
E_PAD = PER_TILE * NSUB
N_CHUNKS = PER_TILE // C
N_ITER = N_CHUNKS // NIB
NA = N + 8
ROWS_PER_TILE = N // NSUB
BLK = 1000

_SC_PARAMS = pltpu.CompilerParams(use_tc_tiling_on_sc=False)


def _pre_body(e_ref, b_ref, s_ref, wbt_ref, bb_ref, wst_ref, bs_ref, cat_ref):
    b = b_ref[...]
    bnorm = jnp.sqrt(jnp.sum(b * b, axis=-1, keepdims=True))
    safe = jnp.maximum(bnorm, 1e-10)
    arg = jnp.minimum(bnorm, 1.0 - 1e-5)
    atanh = 0.5 * jnp.log((1.0 + arg) / (1.0 - arg))
    bt = atanh * b / safe
    bt = jnp.dot(bt, wbt_ref[...], preferred_element_type=jnp.float32) + bb_ref[...]
    s = s_ref[...]
    snorm = jnp.sqrt(jnp.sum(s * s, axis=-1, keepdims=True))
    sn = s / jnp.maximum(snorm, 1e-12)
    st = jnp.dot(sn, wst_ref[...], preferred_element_type=jnp.float32) + bs_ref[...]
    e = e_ref[...]
    cat_ref[0] = e[:, :G]
    cat_ref[1] = jnp.concatenate([e[:, G:], bt[:, : 2 * G - D]], axis=1)
    cat_ref[2] = jnp.concatenate([bt[:, 2 * G - D :], st[:, : 3 * G - 2 * D]], axis=1)
    cat_ref[3] = st[:, 3 * G - 2 * D :]


def _pre(e_emb, b_emb, s_emb, wbt, bb, wst, bs):
    grid = N // BLK
    return pl.pallas_call(
        _pre_body,
        grid=(grid,),
        in_specs=[
            pl.BlockSpec((BLK, D), lambda i: (i, 0)),
            pl.BlockSpec((BLK, D), lambda i: (i, 0)),
            pl.BlockSpec((BLK, D), lambda i: (i, 0)),
            pl.BlockSpec((D, D), lambda i: (0, 0)),
            pl.BlockSpec((1, D), lambda i: (0, 0)),
            pl.BlockSpec((D, D), lambda i: (0, 0)),
            pl.BlockSpec((1, D), lambda i: (0, 0)),
        ],
        out_specs=pl.BlockSpec((4, BLK, G), lambda i: (0, i, 0)),
        out_shape=jax.ShapeDtypeStruct((4, N, G), jnp.float32),
    )(e_emb, b_emb, s_emb, wbt, bb, wst, bs)


def _agg(cat4, srcp, dstp, zf, zd, ones):
    mesh = plsc.VectorSubcoreMesh(core_axis_name="c", subcore_axis_name="s")

    @functools.partial(
        pl.kernel,
        out_type=[
            jax.ShapeDtypeStruct((4, N, G), jnp.float32),
            jax.ShapeDtypeStruct((N, DW), jnp.float32),
        ],
        mesh=mesh,
        scratch_types=[
            pltpu.VMEM((N_CHUNKS, C), jnp.int32),
            [pltpu.VMEM((C,), jnp.int32)] * NIB,
            [pltpu.VMEM((C, G), jnp.float32)] * 2,
            pltpu.VMEM((C, DW), jnp.float32),
            pltpu.VMEM_SHARED((NA, G), jnp.float32),
            pltpu.VMEM_SHARED((NA, DW), jnp.float32),
            [pltpu.SemaphoreType.DMA] * NIB,
            [pltpu.SemaphoreType.DMA] * 2,
        ],
        compiler_params=_SC_PARAMS,
    )
    def agg(cat_hbm, srcp_hbm, dstp_hbm, zf_hbm, zd_hbm, ones_hbm,
            summ_hbm, deg_hbm,
            dst_a, ibufs, bufs, ones_v, acc_sh, deg_sh, isems, gsems):
        c = lax.axis_index("c")
        s = lax.axis_index("s")
        r0 = s * ROWS_PER_TILE

        for g in range(2):
            grp = 2 * c + g
            pltpu.sync_copy(zf_hbm.at[pl.ds(r0, ROWS_PER_TILE)],
                            acc_sh.at[pl.ds(r0, ROWS_PER_TILE)])
            if g == 0:
                pltpu.sync_copy(dstp_hbm.at[s], dst_a)
                pltpu.sync_copy(ones_hbm, ones_v)

                @pl.when(c == 0)
                def _():
                    pltpu.sync_copy(zd_hbm.at[pl.ds(r0, ROWS_PER_TILE)],
                                    deg_sh.at[pl.ds(r0, ROWS_PER_TILE)])

            plsc.subcore_barrier()

            count_deg = g == 0
            for j in range(NIB):
                pltpu.async_copy(srcp_hbm.at[grp, s, j], ibufs[j], isems[j])
            for j in range(2):
                pltpu.make_async_copy(srcp_hbm.at[grp, s, j], ibufs[j],
                                      isems[j]).wait()
                pltpu.async_copy(cat_hbm.at[ibufs[j]], bufs[j], gsems[j])

            def body(k, _):
                i0 = NIB * k
                for j in range(NIB):
                    i = i0 + j
                    b = j % 2
                    islot = (j + 2) % NIB
                    pltpu.make_async_copy(cat_hbm.at[ibufs[j]], bufs[b],
                                          gsems[b]).wait()

                    @pl.when(k < N_ITER - 1)
                    def _():
                        pltpu.async_copy(srcp_hbm.at[grp, s, i + NIB],
                                         ibufs[j], isems[j])

                    pltpu.sync_copy(bufs[b], acc_sh.at[dst_a.at[i]], add=True)
                    if count_deg:
                        @pl.when(c == 0)
                        def _():
                            pltpu.sync_copy(ones_v, deg_sh.at[dst_a.at[i]],
                                            add=True)

                    @pl.when(i + 2 < N_CHUNKS)
                    def _():
                        pltpu.make_async_copy(srcp_hbm.at[grp, s, 0],
                                              ibufs[islot], isems[islot]).wait()
                        pltpu.async_copy(cat_hbm.at[ibufs[islot]], bufs[b],
                                         gsems[b])

                return ()

            lax.fori_loop(0, N_ITER, body, ())
            plsc.subcore_barrier()

            pltpu.sync_copy(acc_sh.at[pl.ds(r0, ROWS_PER_TILE)],
                            summ_hbm.at[grp, pl.ds(r0, ROWS_PER_TILE)])
            if g == 0:
                @pl.when(c == 0)
                def _():
                    pltpu.sync_copy(deg_sh.at[pl.ds(r0, ROWS_PER_TILE)],
                                    deg_hbm.at[pl.ds(r0, ROWS_PER_TILE)])

            plsc.subcore_barrier()

    cat_flat = cat4.reshape(4 * N, G)
    return agg(cat_flat, srcp, dstp, zf, zd, ones)


def _post_body(summ_ref, deg_ref, wet_ref, be_ref, e_ref, b_ref, s_ref):
    d = jnp.maximum(deg_ref[...][:, 0:1], 1.0)
    t0 = summ_ref[0]
    t1 = summ_ref[1]
    t2 = summ_ref[2]
    t3 = summ_ref[3]
    agg_e = jnp.concatenate([t0, t1[:, : D - G]], axis=1) / d
    e = jnp.dot(agg_e, wet_ref[...], preferred_element_type=jnp.float32) + be_ref[...]
    e_ref[...] = jnp.where(e >= 0, e, 0.2 * e)
    agg_b = jnp.concatenate([t1[:, D - G :], t2[:, : 2 * D - 2 * G]], axis=1) / d
    bnorm = jnp.sqrt(jnp.sum(agg_b * agg_b, axis=-1, keepdims=True))
    bsafe = jnp.maximum(bnorm, 1e-10)
    b_ref[...] = jnp.tanh(bnorm) * agg_b / bsafe
    agg_s = jnp.concatenate([t2[:, 2 * D - 2 * G :], t3], axis=1) / d
    snorm = jnp.sqrt(jnp.sum(agg_s * agg_s, axis=-1, keepdims=True))
    s_ref[...] = agg_s / jnp.maximum(snorm, 1e-12)


def _post(summ, deg, wet, be):
    grid = N // BLK
    return pl.pallas_call(
        _post_body,
        grid=(grid,),
        in_specs=[
            pl.BlockSpec((4, BLK, G), lambda i: (0, i, 0)),
            pl.BlockSpec((BLK, DW), lambda i: (i, 0)),
            pl.BlockSpec((D, D), lambda i: (0, 0)),
            pl.BlockSpec((1, D), lambda i: (0, 0)),
        ],
        out_specs=[
            pl.BlockSpec((BLK, D), lambda i: (i, 0)),
            pl.BlockSpec((BLK, D), lambda i: (i, 0)),
            pl.BlockSpec((BLK, D), lambda i: (i, 0)),
        ],
        out_shape=[
            jax.ShapeDtypeStruct((N, D), jnp.float32),
            jax.ShapeDtypeStruct((N, D), jnp.float32),
            jax.ShapeDtypeStruct((N, D), jnp.float32),
        ],
    )(summ, deg, wet, be)


def kernel(e_emb, b_emb, s_emb, edge_index, W_e, b_e, W_b, b_b, W_s, b_s):
    pad = E_PAD - E
    src = jnp.concatenate([edge_index[0], jnp.zeros((pad,), jnp.int32)])
    dst = jnp.concatenate([edge_index[1], jnp.full((pad,), N, jnp.int32)])
    srcp = jnp.stack([src, src + N, src + 2 * N, src + 3 * N])
    srcp = srcp.reshape(4, NSUB, N_CHUNKS, C)
    dstp = dst.reshape(NSUB, N_CHUNKS, C)
    zf = jnp.zeros((N, G), jnp.float32)
    zd = jnp.zeros((N, DW), jnp.float32)
    ones = jnp.ones((C, DW), jnp.float32)

    cat4 = _pre(e_emb, b_emb, s_emb,
                W_b.T, b_b.reshape(1, D), W_s.T, b_s.reshape(1, D))
    summ, deg = _agg(cat4, srcp, dstp, zf, zd, ones)
    e_out, b_out, s_out = _post(summ, deg, W_e.T, b_e.reshape(1, D))
    return (e_out, b_out, s_out)


# speedup vs baseline: 1.8483x; 1.8483x over previous
import functools

import jax
import jax.numpy as jnp
from jax import lax
from jax.experimental import pallas as pl
from jax.experimental.pallas import tpu as pltpu
from jax.experimental.pallas import tpu_sc as plsc

N = 10000
E = 320000
D = 128
G = 96
DW = 16
C = 80
NSUB = 16
PER_TILE = E // NSUB
N_CHUNKS = PER_TILE // C
N_HALF = N_CHUNKS // 2
ROWS_PER_TILE = N // NSUB
BLK = 1000

_SC_PARAMS = pltpu.CompilerParams(use_tc_tiling_on_sc=False)


def _pre_body(e_ref, b_ref, s_ref, wbt_ref, bb_ref, wst_ref, bs_ref, cat_ref):
    b = b_ref[...]
    bnorm = jnp.sqrt(jnp.sum(b * b, axis=-1, keepdims=True))
    safe = jnp.maximum(bnorm, 1e-10)
    arg = jnp.minimum(bnorm, 1.0 - 1e-5)
    atanh = 0.5 * jnp.log((1.0 + arg) / (1.0 - arg))
    bt = atanh * b / safe
    bt = jnp.dot(bt, wbt_ref[...], preferred_element_type=jnp.float32) + bb_ref[...]
    s = s_ref[...]
    snorm = jnp.sqrt(jnp.sum(s * s, axis=-1, keepdims=True))
    sn = s / jnp.maximum(snorm, 1e-12)
    st = jnp.dot(sn, wst_ref[...], preferred_element_type=jnp.float32) + bs_ref[...]
    e = e_ref[...]
    cat_ref[0] = e[:, :G]
    cat_ref[1] = jnp.concatenate([e[:, G:], bt[:, : 2 * G - D]], axis=1)
    cat_ref[2] = jnp.concatenate([bt[:, 2 * G - D :], st[:, : 3 * G - 2 * D]], axis=1)
    cat_ref[3] = st[:, 3 * G - 2 * D :]


def _pre(e_emb, b_emb, s_emb, wbt, bb, wst, bs):
    grid = N // BLK
    return pl.pallas_call(
        _pre_body,
        grid=(grid,),
        in_specs=[
            pl.BlockSpec((BLK, D), lambda i: (i, 0)),
            pl.BlockSpec((BLK, D), lambda i: (i, 0)),
            pl.BlockSpec((BLK, D), lambda i: (i, 0)),
            pl.BlockSpec((D, D), lambda i: (0, 0)),
            pl.BlockSpec((1, D), lambda i: (0, 0)),
            pl.BlockSpec((D, D), lambda i: (0, 0)),
            pl.BlockSpec((1, D), lambda i: (0, 0)),
        ],
        out_specs=pl.BlockSpec((4, BLK, G), lambda i: (0, i, 0)),
        out_shape=jax.ShapeDtypeStruct((4, N, G), jnp.float32),
    )(e_emb, b_emb, s_emb, wbt, bb, wst, bs)


def _agg(cat4, srcp, dstp, zf, zd, ones):
    mesh = plsc.VectorSubcoreMesh(core_axis_name="c", subcore_axis_name="s")

    @functools.partial(
        pl.kernel,
        out_type=[
            jax.ShapeDtypeStruct((4, N, G), jnp.float32),
            jax.ShapeDtypeStruct((N, DW), jnp.float32),
        ],
        mesh=mesh,
        scratch_types=[
            pltpu.VMEM((N_CHUNKS, C), jnp.int32),
            pltpu.VMEM((N_CHUNKS, C), jnp.int32),
            pltpu.VMEM((C, G), jnp.float32),
            pltpu.VMEM((C, G), jnp.float32),
            pltpu.VMEM((C, DW), jnp.float32),
            pltpu.VMEM_SHARED((N, G), jnp.float32),
            pltpu.VMEM_SHARED((N, DW), jnp.float32),
            pltpu.SemaphoreType.DMA,
            pltpu.SemaphoreType.DMA,
        ],
        compiler_params=_SC_PARAMS,
    )
    def agg(cat_hbm, srcp_hbm, dstp_hbm, zf_hbm, zd_hbm, ones_hbm,
            summ_hbm, deg_hbm,
            src_a, dst_a, buf0, buf1, ones_v, acc_sh, deg_sh, sem0, sem1):
        c = lax.axis_index("c")
        s = lax.axis_index("s")
        r0 = s * ROWS_PER_TILE

        for g in range(2):
            grp = 2 * c + g
            pltpu.sync_copy(zf_hbm.at[pl.ds(r0, ROWS_PER_TILE)],
                            acc_sh.at[pl.ds(r0, ROWS_PER_TILE)])
            if g == 0:
                pltpu.sync_copy(dstp_hbm.at[s], dst_a)
                pltpu.sync_copy(ones_hbm, ones_v)

                @pl.when(c == 0)
                def _():
                    pltpu.sync_copy(zd_hbm.at[pl.ds(r0, ROWS_PER_TILE)],
                                    deg_sh.at[pl.ds(r0, ROWS_PER_TILE)])

            pltpu.sync_copy(srcp_hbm.at[grp, s], src_a)
            plsc.subcore_barrier()

            count_deg = g == 0
            pltpu.async_copy(cat_hbm.at[src_a.at[0]], buf0, sem0)

            def body(k, _):
                i0 = 2 * k
                pltpu.async_copy(cat_hbm.at[src_a.at[i0 + 1]], buf1, sem1)
                pltpu.make_async_copy(cat_hbm.at[src_a.at[i0]], buf0, sem0).wait()
                pltpu.sync_copy(buf0, acc_sh.at[dst_a.at[i0]], add=True)
                if count_deg:
                    @pl.when(c == 0)
                    def _():
                        pltpu.sync_copy(ones_v, deg_sh.at[dst_a.at[i0]], add=True)

                @pl.when(k < N_HALF - 1)
                def _():
                    pltpu.async_copy(cat_hbm.at[src_a.at[i0 + 2]], buf0, sem0)

                pltpu.make_async_copy(cat_hbm.at[src_a.at[i0 + 1]], buf1, sem1).wait()
                pltpu.sync_copy(buf1, acc_sh.at[dst_a.at[i0 + 1]], add=True)
                if count_deg:
                    @pl.when(c == 0)
                    def _():
                        pltpu.sync_copy(ones_v, deg_sh.at[dst_a.at[i0 + 1]],
                                        add=True)

                return ()

            lax.fori_loop(0, N_HALF, body, ())
            plsc.subcore_barrier()

            pltpu.sync_copy(acc_sh.at[pl.ds(r0, ROWS_PER_TILE)],
                            summ_hbm.at[grp, pl.ds(r0, ROWS_PER_TILE)])
            if g == 0:
                @pl.when(c == 0)
                def _():
                    pltpu.sync_copy(deg_sh.at[pl.ds(r0, ROWS_PER_TILE)],
                                    deg_hbm.at[pl.ds(r0, ROWS_PER_TILE)])

            plsc.subcore_barrier()

    cat_flat = cat4.reshape(4 * N, G)
    return agg(cat_flat, srcp, dstp, zf, zd, ones)


def _post_body(summ_ref, deg_ref, wet_ref, be_ref, e_ref, b_ref, s_ref):
    d = jnp.maximum(deg_ref[...][:, 0:1], 1.0)
    t0 = summ_ref[0]
    t1 = summ_ref[1]
    t2 = summ_ref[2]
    t3 = summ_ref[3]
    agg_e = jnp.concatenate([t0, t1[:, : D - G]], axis=1) / d
    e = jnp.dot(agg_e, wet_ref[...], preferred_element_type=jnp.float32) + be_ref[...]
    e_ref[...] = jnp.where(e >= 0, e, 0.2 * e)
    agg_b = jnp.concatenate([t1[:, D - G :], t2[:, : 2 * D - 2 * G]], axis=1) / d
    bnorm = jnp.sqrt(jnp.sum(agg_b * agg_b, axis=-1, keepdims=True))
    bsafe = jnp.maximum(bnorm, 1e-10)
    b_ref[...] = jnp.tanh(bnorm) * agg_b / bsafe
    agg_s = jnp.concatenate([t2[:, 2 * D - 2 * G :], t3], axis=1) / d
    snorm = jnp.sqrt(jnp.sum(agg_s * agg_s, axis=-1, keepdims=True))
    s_ref[...] = agg_s / jnp.maximum(snorm, 1e-12)


def _post(summ, deg, wet, be):
    grid = N // BLK
    return pl.pallas_call(
        _post_body,
        grid=(grid,),
        in_specs=[
            pl.BlockSpec((4, BLK, G), lambda i: (0, i, 0)),
            pl.BlockSpec((BLK, DW), lambda i: (i, 0)),
            pl.BlockSpec((D, D), lambda i: (0, 0)),
            pl.BlockSpec((1, D), lambda i: (0, 0)),
        ],
        out_specs=[
            pl.BlockSpec((BLK, D), lambda i: (i, 0)),
            pl.BlockSpec((BLK, D), lambda i: (i, 0)),
            pl.BlockSpec((BLK, D), lambda i: (i, 0)),
        ],
        out_shape=[
            jax.ShapeDtypeStruct((N, D), jnp.float32),
            jax.ShapeDtypeStruct((N, D), jnp.float32),
            jax.ShapeDtypeStruct((N, D), jnp.float32),
        ],
    )(summ, deg, wet, be)


def kernel(e_emb, b_emb, s_emb, edge_index, W_e, b_e, W_b, b_b, W_s, b_s):
    src = edge_index[0]
    dst = edge_index[1]
    srcp = jnp.stack([src, src + N, src + 2 * N, src + 3 * N])
    srcp = srcp.reshape(4, NSUB, N_CHUNKS, C)
    dstp = dst.reshape(NSUB, N_CHUNKS, C)
    zf = jnp.zeros((N, G), jnp.float32)
    zd = jnp.zeros((N, DW), jnp.float32)
    ones = jnp.ones((C, DW), jnp.float32)

    cat4 = _pre(e_emb, b_emb, s_emb,
                W_b.T, b_b.reshape(1, D), W_s.T, b_s.reshape(1, D))
    summ, deg = _agg(cat4, srcp, dstp, zf, zd, ones)
    e_out, b_out, s_out = _post(summ, deg, W_e.T, b_e.reshape(1, D))
    return (e_out, b_out, s_out)


# speedup vs baseline: 1.8906x; 1.0229x over previous
import functools

import jax
import jax.numpy as jnp
from jax import lax
from jax.experimental import pallas as pl
from jax.experimental.pallas import tpu as pltpu
from jax.experimental.pallas import tpu_sc as plsc

N = 10000
E = 320000
D = 128
G = 96
DW = 16
C = 80
NSUB = 16
PER_TILE = E // NSUB
N_CHUNKS = PER_TILE // C
N_HALF = N_CHUNKS // 2
ROWS_PER_TILE = N // NSUB
BLK = 1000

_SC_PARAMS = pltpu.CompilerParams(use_tc_tiling_on_sc=False)


def _pre_body(e_ref, b_ref, s_ref, wbt_ref, bb_ref, wst_ref, bs_ref, cat_ref):
    b = b_ref[...]
    bnorm = jnp.sqrt(jnp.sum(b * b, axis=-1, keepdims=True))
    safe = jnp.maximum(bnorm, 1e-10)
    arg = jnp.minimum(bnorm, 1.0 - 1e-5)
    atanh = 0.5 * jnp.log((1.0 + arg) / (1.0 - arg))
    bt = atanh * b / safe
    bt = jnp.dot(bt, wbt_ref[...], preferred_element_type=jnp.float32) + bb_ref[...]
    s = s_ref[...]
    snorm = jnp.sqrt(jnp.sum(s * s, axis=-1, keepdims=True))
    sn = s / jnp.maximum(snorm, 1e-12)
    st = jnp.dot(sn, wst_ref[...], preferred_element_type=jnp.float32) + bs_ref[...]
    e = e_ref[...]
    cat_ref[0] = e[:, :G]
    cat_ref[1] = jnp.concatenate([e[:, G:], bt[:, : 2 * G - D]], axis=1)
    cat_ref[2] = jnp.concatenate([bt[:, 2 * G - D :], st[:, : 3 * G - 2 * D]], axis=1)
    cat_ref[3] = st[:, 3 * G - 2 * D :]


def _pre(e_emb, b_emb, s_emb, wbt, bb, wst, bs):
    grid = N // BLK
    return pl.pallas_call(
        _pre_body,
        grid=(grid,),
        in_specs=[
            pl.BlockSpec((BLK, D), lambda i: (i, 0)),
            pl.BlockSpec((BLK, D), lambda i: (i, 0)),
            pl.BlockSpec((BLK, D), lambda i: (i, 0)),
            pl.BlockSpec((D, D), lambda i: (0, 0)),
            pl.BlockSpec((1, D), lambda i: (0, 0)),
            pl.BlockSpec((D, D), lambda i: (0, 0)),
            pl.BlockSpec((1, D), lambda i: (0, 0)),
        ],
        out_specs=pl.BlockSpec((4, BLK, G), lambda i: (0, i, 0)),
        out_shape=jax.ShapeDtypeStruct((4, N, G), jnp.float32),
    )(e_emb, b_emb, s_emb, wbt, bb, wst, bs)


def _agg(cat4, srcp, dstp, zf, zd, ones):
    mesh = plsc.VectorSubcoreMesh(core_axis_name="c", subcore_axis_name="s")

    @functools.partial(
        pl.kernel,
        out_type=[
            jax.ShapeDtypeStruct((4, N, G), jnp.float32),
            jax.ShapeDtypeStruct((N, DW), jnp.float32),
        ],
        mesh=mesh,
        scratch_types=[
            pltpu.VMEM((N_CHUNKS, C), jnp.int32),
            pltpu.VMEM((N_CHUNKS, C), jnp.int32),
            pltpu.VMEM((C, G), jnp.float32),
            pltpu.VMEM((C, G), jnp.float32),
            pltpu.VMEM((C, DW), jnp.float32),
            pltpu.VMEM_SHARED((N, G), jnp.float32),
            pltpu.VMEM_SHARED((N, DW), jnp.float32),
            pltpu.SemaphoreType.DMA,
            pltpu.SemaphoreType.DMA,
            pltpu.SemaphoreType.DMA,
        ],
        compiler_params=_SC_PARAMS,
    )
    def agg(cat_hbm, srcp_hbm, dstp_hbm, zf_hbm, zd_hbm, ones_hbm,
            summ_hbm, deg_hbm,
            src_a, dst_a, buf0, buf1, ones_v, acc_sh, deg_sh, sem0, sem1, osem):
        c = lax.axis_index("c")
        s = lax.axis_index("s")
        r0 = s * ROWS_PER_TILE

        for g in range(2):
            grp = 2 * c + g
            pltpu.sync_copy(zf_hbm.at[pl.ds(r0, ROWS_PER_TILE)],
                            acc_sh.at[pl.ds(r0, ROWS_PER_TILE)])
            if g == 0:
                pltpu.sync_copy(dstp_hbm.at[s], dst_a)
                pltpu.sync_copy(ones_hbm, ones_v)

                @pl.when(c == 0)
                def _():
                    pltpu.sync_copy(zd_hbm.at[pl.ds(r0, ROWS_PER_TILE)],
                                    deg_sh.at[pl.ds(r0, ROWS_PER_TILE)])

            pltpu.sync_copy(srcp_hbm.at[grp, s], src_a)
            plsc.subcore_barrier()

            count_deg = g == 0
            pltpu.async_copy(cat_hbm.at[src_a.at[0]], buf0, sem0)

            def body(k, _):
                i0 = 2 * k
                pltpu.async_copy(cat_hbm.at[src_a.at[i0 + 1]], buf1, sem1)
                pltpu.make_async_copy(cat_hbm.at[src_a.at[i0]], buf0, sem0).wait()
                pltpu.sync_copy(buf0, acc_sh.at[dst_a.at[i0]], add=True)
                if count_deg:
                    @pl.when((c == 0) & (k > 0))
                    def _():
                        pltpu.make_async_copy(ones_v, deg_sh.at[dst_a.at[i0]],
                                              osem).wait()

                    @pl.when(c == 0)
                    def _():
                        pltpu.async_copy(ones_v, deg_sh.at[dst_a.at[i0]],
                                         osem, add=True)

                @pl.when(k < N_HALF - 1)
                def _():
                    pltpu.async_copy(cat_hbm.at[src_a.at[i0 + 2]], buf0, sem0)

                pltpu.make_async_copy(cat_hbm.at[src_a.at[i0 + 1]], buf1, sem1).wait()
                pltpu.sync_copy(buf1, acc_sh.at[dst_a.at[i0 + 1]], add=True)
                if count_deg:
                    @pl.when(c == 0)
                    def _():
                        pltpu.make_async_copy(ones_v, deg_sh.at[dst_a.at[i0]],
                                              osem).wait()
                        pltpu.async_copy(ones_v, deg_sh.at[dst_a.at[i0 + 1]],
                                         osem, add=True)

                return ()

            lax.fori_loop(0, N_HALF, body, ())
            if count_deg:
                @pl.when(c == 0)
                def _():
                    pltpu.make_async_copy(ones_v, deg_sh.at[dst_a.at[0]],
                                          osem).wait()

            plsc.subcore_barrier()

            pltpu.sync_copy(acc_sh.at[pl.ds(r0, ROWS_PER_TILE)],
                            summ_hbm.at[grp, pl.ds(r0, ROWS_PER_TILE)])
            if g == 0:
                @pl.when(c == 0)
                def _():
                    pltpu.sync_copy(deg_sh.at[pl.ds(r0, ROWS_PER_TILE)],
                                    deg_hbm.at[pl.ds(r0, ROWS_PER_TILE)])

            plsc.subcore_barrier()

    cat_flat = cat4.reshape(4 * N, G)
    return agg(cat_flat, srcp, dstp, zf, zd, ones)


def _post_body(summ_ref, deg_ref, wet_ref, be_ref, e_ref, b_ref, s_ref):
    d = jnp.maximum(deg_ref[...][:, 0:1], 1.0)
    t0 = summ_ref[0]
    t1 = summ_ref[1]
    t2 = summ_ref[2]
    t3 = summ_ref[3]
    agg_e = jnp.concatenate([t0, t1[:, : D - G]], axis=1) / d
    e = jnp.dot(agg_e, wet_ref[...], preferred_element_type=jnp.float32) + be_ref[...]
    e_ref[...] = jnp.where(e >= 0, e, 0.2 * e)
    agg_b = jnp.concatenate([t1[:, D - G :], t2[:, : 2 * D - 2 * G]], axis=1) / d
    bnorm = jnp.sqrt(jnp.sum(agg_b * agg_b, axis=-1, keepdims=True))
    bsafe = jnp.maximum(bnorm, 1e-10)
    b_ref[...] = jnp.tanh(bnorm) * agg_b / bsafe
    agg_s = jnp.concatenate([t2[:, 2 * D - 2 * G :], t3], axis=1) / d
    snorm = jnp.sqrt(jnp.sum(agg_s * agg_s, axis=-1, keepdims=True))
    s_ref[...] = agg_s / jnp.maximum(snorm, 1e-12)


def _post(summ, deg, wet, be):
    grid = N // BLK
    return pl.pallas_call(
        _post_body,
        grid=(grid,),
        in_specs=[
            pl.BlockSpec((4, BLK, G), lambda i: (0, i, 0)),
            pl.BlockSpec((BLK, DW), lambda i: (i, 0)),
            pl.BlockSpec((D, D), lambda i: (0, 0)),
            pl.BlockSpec((1, D), lambda i: (0, 0)),
        ],
        out_specs=[
            pl.BlockSpec((BLK, D), lambda i: (i, 0)),
            pl.BlockSpec((BLK, D), lambda i: (i, 0)),
            pl.BlockSpec((BLK, D), lambda i: (i, 0)),
        ],
        out_shape=[
            jax.ShapeDtypeStruct((N, D), jnp.float32),
            jax.ShapeDtypeStruct((N, D), jnp.float32),
            jax.ShapeDtypeStruct((N, D), jnp.float32),
        ],
    )(summ, deg, wet, be)


def kernel(e_emb, b_emb, s_emb, edge_index, W_e, b_e, W_b, b_b, W_s, b_s):
    src = edge_index[0]
    dst = edge_index[1]
    srcp = jnp.stack([src, src + N, src + 2 * N, src + 3 * N])
    srcp = srcp.reshape(4, NSUB, N_CHUNKS, C)
    dstp = dst.reshape(NSUB, N_CHUNKS, C)
    zf = jnp.zeros((N, G), jnp.float32)
    zd = jnp.zeros((N, DW), jnp.float32)
    ones = jnp.ones((C, DW), jnp.float32)

    cat4 = _pre(e_emb, b_emb, s_emb,
                W_b.T, b_b.reshape(1, D), W_s.T, b_s.reshape(1, D))
    summ, deg = _agg(cat4, srcp, dstp, zf, zd, ones)
    e_out, b_out, s_out = _post(summ, deg, W_e.T, b_e.reshape(1, D))
    return (e_out, b_out, s_out)
